# baseline (speedup 1.0000x reference)
import functools

import jax
import jax.numpy as jnp
from jax import lax
from jax.experimental import pallas as pl
from jax.experimental.pallas import tpu as pltpu
from jax.experimental.pallas import tpu_sc as plsc

N_NODES = 10000
N_EDGES = 320000
D = 128

NC = 2
NS = 16
W = 50
NB = 4
EDGES_PER_TILE = N_EDGES // (NC * NS)
WINDOWS_PER_TILE = EDGES_PER_TILE // W
NPH = 5
WPP = WINDOWS_PER_TILE // NPH
ROWCHUNK = 40
N_ROW_CHUNKS = N_NODES // ROWCHUNK
CHUNKS_PER_SUBCORE = -(-N_ROW_CHUNKS // NS)


def _sc_aggregate(x, zeros_rows, src2d, dst2d):
    mesh = plsc.VectorSubcoreMesh(core_axis_name="c", subcore_axis_name="s")

    @functools.partial(
        pl.kernel,
        out_type=jax.ShapeDtypeStruct((NC, N_NODES, D), jnp.float32),
        mesh=mesh,
        scratch_types=[
            pltpu.VMEM((WPP, W), jnp.int32),
            pltpu.VMEM((WPP, W), jnp.int32),
            pltpu.VMEM((W, D), jnp.float32),
            pltpu.VMEM((W, D), jnp.float32),
            pltpu.VMEM((W, D), jnp.float32),
            pltpu.VMEM((W, D), jnp.float32),
            pltpu.SemaphoreType.DMA,
            pltpu.SemaphoreType.DMA,
            pltpu.SemaphoreType.DMA,
            pltpu.SemaphoreType.DMA,
            pltpu.SemaphoreType.DMA,
            pltpu.SemaphoreType.DMA,
            pltpu.SemaphoreType.DMA,
            pltpu.SemaphoreType.DMA,
            pltpu.VMEM_SHARED((N_NODES, D), jnp.float32),
        ],
    )
    def k(x_hbm, z_hbm, src_hbm, dst_hbm, out_hbm, src_v, dst_v, rows_0,
          rows_1, rows_2, rows_3, gsem_0, gsem_1, gsem_2, gsem_3,
          ssem_0, ssem_1, ssem_2, ssem_3, acc):
        rows_v = rows_0
        cid = lax.axis_index("c")
        sid = lax.axis_index("s")

        @pl.loop(0, CHUNKS_PER_SUBCORE)
        def _(j):
            k = sid + j * NS

            @pl.when(k < N_ROW_CHUNKS)
            def _():
                pltpu.sync_copy(z_hbm, acc.at[pl.ds(k * ROWCHUNK, ROWCHUNK)])

        wid = cid * NS + sid

        plsc.subcore_barrier()

        def gather_start(w, buf, gsem):
            pltpu.async_copy(x_hbm.at[src_v.at[w]], buf, gsem)

        def gather_wait(w, buf, gsem):
            pltpu.make_async_copy(x_hbm.at[src_v.at[w]], buf, gsem).wait()

        def scatter_start(w, buf, ssem):
            pltpu.async_copy(buf, acc.at[dst_v.at[w]], ssem, add=True)

        def scatter_wait(w, buf, ssem):
            pltpu.make_async_copy(buf, acc.at[dst_v.at[w]], ssem).wait()

        bufs = (
            (rows_0, gsem_0, ssem_0),
            (rows_1, gsem_1, ssem_1),
            (rows_2, gsem_2, ssem_2),
            (rows_3, gsem_3, ssem_3),
        )

        @pl.loop(0, NPH)
        def _(p):
            pltpu.sync_copy(src_hbm.at[wid, p], src_v)
            pltpu.sync_copy(dst_hbm.at[wid, p], dst_v)
            for i in range(2):
                gather_start(i, bufs[i][0], bufs[i][1])

            @pl.loop(0, WPP // NB)
            def _(h):
                w0 = h * NB
                for b in range(NB):
                    w = w0 + b
                    buf, gsem, ssem = bufs[b]
                    nbuf, ngsem, nssem = bufs[(b + 2) % NB]

                    @pl.when(w - 2 >= 0)
                    def _():
                        scatter_wait(w - 2, nbuf, nssem)

                    @pl.when(w + 2 < WPP)
                    def _():
                        gather_start(w + 2, nbuf, ngsem)

                    gather_wait(w, buf, gsem)
                    scatter_start(w, buf, ssem)

            for w in (WPP - 2, WPP - 1):
                buf, _gsem, ssem = bufs[w % NB]
                scatter_wait(w, buf, ssem)

        plsc.subcore_barrier()

        @pl.loop(0, CHUNKS_PER_SUBCORE)
        def _(j):
            k = sid + j * NS

            @pl.when(k < N_ROW_CHUNKS)
            def _():
                base = k * ROWCHUNK
                stage = rows_v.at[pl.ds(0, ROWCHUNK)]
                pltpu.sync_copy(acc.at[pl.ds(base, ROWCHUNK)], stage)
                pltpu.sync_copy(stage, out_hbm.at[cid, pl.ds(base, ROWCHUNK)])

    return k(x, zeros_rows, src2d, dst2d)


def _tc_linear1(x, W1T, b):
    BLK = 1000

    def body(x_ref, w1_ref, b_ref, o_ref):
        o_ref[...] = (
            jnp.dot(x_ref[...], w1_ref[...], preferred_element_type=jnp.float32)
            + b_ref[...]
        )

    return pl.pallas_call(
        body,
        grid=(N_NODES // BLK,),
        in_specs=[
            pl.BlockSpec((BLK, D), lambda i: (i, 0)),
            pl.BlockSpec((D, D), lambda i: (0, 0)),
            pl.BlockSpec((1, D), lambda i: (0, 0)),
        ],
        out_specs=pl.BlockSpec((BLK, D), lambda i: (i, 0)),
        out_shape=jax.ShapeDtypeStruct((N_NODES, D), jnp.float32),
    )(x, W1T, b)


def _tc_combine(y1, partials, W2T):
    BLK = 1000

    def body(y1_ref, p_ref, w2_ref, o_ref):
        agg = p_ref[0] + p_ref[1]
        o_ref[...] = y1_ref[...] + jnp.dot(
            agg, w2_ref[...], preferred_element_type=jnp.float32
        )

    return pl.pallas_call(
        body,
        grid=(N_NODES // BLK,),
        in_specs=[
            pl.BlockSpec((BLK, D), lambda i: (i, 0)),
            pl.BlockSpec((NC, BLK, D), lambda i: (0, i, 0)),
            pl.BlockSpec((D, D), lambda i: (0, 0)),
        ],
        out_specs=pl.BlockSpec((BLK, D), lambda i: (i, 0)),
        out_shape=jax.ShapeDtypeStruct((N_NODES, D), jnp.float32),
    )(y1, partials, W2T)


def kernel(shape_features, edge_index, W1, b1, W2, b2):
    src2d = edge_index[0].reshape(NC * NS, NPH, WPP, W)
    dst2d = edge_index[1].reshape(NC * NS, NPH, WPP, W)
    zeros_rows = jnp.zeros((ROWCHUNK, D), jnp.float32)
    partials = _sc_aggregate(shape_features, zeros_rows, src2d, dst2d)
    b = (b1 + b2).reshape(1, D)
    y1 = _tc_linear1(shape_features, W1.T, b)
    return _tc_combine(y1, partials, W2.T)


# speedup vs baseline: 1.0555x; 1.0555x over previous
import functools

import jax
import jax.numpy as jnp
from jax import lax
from jax.experimental import pallas as pl
from jax.experimental.pallas import tpu as pltpu
from jax.experimental.pallas import tpu_sc as plsc

N_NODES = 10000
N_EDGES = 320000
D = 128

NC = 2
NS = 16
W = 80
NB = 4
EDGES_PER_TILE = N_EDGES // (NC * NS)
WINDOWS_PER_TILE = EDGES_PER_TILE // W
NPH = 5
WPP = WINDOWS_PER_TILE // NPH
ROWCHUNK = 80
N_ROW_CHUNKS = N_NODES // ROWCHUNK
CHUNKS_PER_SUBCORE = -(-N_ROW_CHUNKS // NS)


def _sc_aggregate(x, zeros_rows, src2d, dst2d):
    mesh = plsc.VectorSubcoreMesh(core_axis_name="c", subcore_axis_name="s")

    @functools.partial(
        pl.kernel,
        out_type=jax.ShapeDtypeStruct((NC, N_NODES, D), jnp.float32),
        mesh=mesh,
        scratch_types=[
            pltpu.VMEM((WPP, W), jnp.int32),
            pltpu.VMEM((WPP, W), jnp.int32),
            pltpu.VMEM((W, D), jnp.float32),
            pltpu.VMEM((W, D), jnp.float32),
            pltpu.VMEM((W, D), jnp.float32),
            pltpu.VMEM((W, D), jnp.float32),
            pltpu.SemaphoreType.DMA,
            pltpu.SemaphoreType.DMA,
            pltpu.SemaphoreType.DMA,
            pltpu.SemaphoreType.DMA,
            pltpu.SemaphoreType.DMA,
            pltpu.SemaphoreType.DMA,
            pltpu.SemaphoreType.DMA,
            pltpu.SemaphoreType.DMA,
            pltpu.VMEM_SHARED((N_NODES, D), jnp.float32),
        ],
    )
    def k(x_hbm, z_hbm, src_hbm, dst_hbm, out_hbm, src_v, dst_v, rows_0,
          rows_1, rows_2, rows_3, gsem_0, gsem_1, gsem_2, gsem_3,
          ssem_0, ssem_1, ssem_2, ssem_3, acc):
        rows_v = rows_0
        cid = lax.axis_index("c")
        sid = lax.axis_index("s")

        @pl.loop(0, CHUNKS_PER_SUBCORE)
        def _(j):
            k = sid + j * NS

            @pl.when(k < N_ROW_CHUNKS)
            def _():
                pltpu.sync_copy(z_hbm, acc.at[pl.ds(k * ROWCHUNK, ROWCHUNK)])

        wid = cid * NS + sid

        plsc.subcore_barrier()

        def gather_start(w, buf, gsem):
            pltpu.async_copy(x_hbm.at[src_v.at[w]], buf, gsem)

        def gather_wait(w, buf, gsem):
            pltpu.make_async_copy(x_hbm.at[src_v.at[w]], buf, gsem).wait()

        def scatter_start(w, buf, ssem):
            pltpu.async_copy(buf, acc.at[dst_v.at[w]], ssem, add=True)

        def scatter_wait(w, buf, ssem):
            pltpu.make_async_copy(buf, acc.at[dst_v.at[w]], ssem).wait()

        bufs = (
            (rows_0, gsem_0, ssem_0),
            (rows_1, gsem_1, ssem_1),
            (rows_2, gsem_2, ssem_2),
            (rows_3, gsem_3, ssem_3),
        )

        @pl.loop(0, NPH)
        def _(p):
            pltpu.sync_copy(src_hbm.at[wid, p], src_v)
            pltpu.sync_copy(dst_hbm.at[wid, p], dst_v)
            for i in range(2):
                gather_start(i, bufs[i][0], bufs[i][1])

            @pl.loop(0, WPP // NB)
            def _(h):
                w0 = h * NB
                for b in range(NB):
                    w = w0 + b
                    buf, gsem, ssem = bufs[b]
                    nbuf, ngsem, nssem = bufs[(b + 2) % NB]

                    @pl.when(w - 2 >= 0)
                    def _():
                        scatter_wait(w - 2, nbuf, nssem)

                    @pl.when(w + 2 < WPP)
                    def _():
                        gather_start(w + 2, nbuf, ngsem)

                    gather_wait(w, buf, gsem)
                    scatter_start(w, buf, ssem)

            wt = WPP - 1
            buf_t, gsem_t, ssem_t = bufs[wt % NB]
            _nb_t, _ng_t, nssem_t = bufs[(wt + 2) % NB]
            scatter_wait(wt - 2, _nb_t, nssem_t)
            gather_wait(wt, buf_t, gsem_t)
            scatter_start(wt, buf_t, ssem_t)

            for wd in (WPP - 2, WPP - 1):
                buf_d, _gsem_d, ssem_d = bufs[wd % NB]
                scatter_wait(wd, buf_d, ssem_d)

        plsc.subcore_barrier()

        @pl.loop(0, CHUNKS_PER_SUBCORE)
        def _(j):
            k = sid + j * NS

            @pl.when(k < N_ROW_CHUNKS)
            def _():
                base = k * ROWCHUNK
                stage = rows_v.at[pl.ds(0, ROWCHUNK)]
                pltpu.sync_copy(acc.at[pl.ds(base, ROWCHUNK)], stage)
                pltpu.sync_copy(stage, out_hbm.at[cid, pl.ds(base, ROWCHUNK)])

    return k(x, zeros_rows, src2d, dst2d)


def _tc_linear1(x, W1T, b):
    BLK = 1000

    def body(x_ref, w1_ref, b_ref, o_ref):
        o_ref[...] = (
            jnp.dot(x_ref[...], w1_ref[...], preferred_element_type=jnp.float32)
            + b_ref[...]
        )

    return pl.pallas_call(
        body,
        grid=(N_NODES // BLK,),
        in_specs=[
            pl.BlockSpec((BLK, D), lambda i: (i, 0)),
            pl.BlockSpec((D, D), lambda i: (0, 0)),
            pl.BlockSpec((1, D), lambda i: (0, 0)),
        ],
        out_specs=pl.BlockSpec((BLK, D), lambda i: (i, 0)),
        out_shape=jax.ShapeDtypeStruct((N_NODES, D), jnp.float32),
    )(x, W1T, b)


def _tc_combine(y1, partials, W2T):
    BLK = 1000

    def body(y1_ref, p_ref, w2_ref, o_ref):
        agg = p_ref[0] + p_ref[1]
        o_ref[...] = y1_ref[...] + jnp.dot(
            agg, w2_ref[...], preferred_element_type=jnp.float32
        )

    return pl.pallas_call(
        body,
        grid=(N_NODES // BLK,),
        in_specs=[
            pl.BlockSpec((BLK, D), lambda i: (i, 0)),
            pl.BlockSpec((NC, BLK, D), lambda i: (0, i, 0)),
            pl.BlockSpec((D, D), lambda i: (0, 0)),
        ],
        out_specs=pl.BlockSpec((BLK, D), lambda i: (i, 0)),
        out_shape=jax.ShapeDtypeStruct((N_NODES, D), jnp.float32),
    )(y1, partials, W2T)


def kernel(shape_features, edge_index, W1, b1, W2, b2):
    src2d = edge_index[0].reshape(NC * NS, NPH, WPP, W)
    dst2d = edge_index[1].reshape(NC * NS, NPH, WPP, W)
    zeros_rows = jnp.zeros((ROWCHUNK, D), jnp.float32)
    partials = _sc_aggregate(shape_features, zeros_rows, src2d, dst2d)
    b = (b1 + b2).reshape(1, D)
    y1 = _tc_linear1(shape_features, W1.T, b)
    return _tc_combine(y1, partials, W2.T)


# speedup vs baseline: 1.1424x; 1.0823x over previous
import functools

import jax
import jax.numpy as jnp
from jax import lax
from jax.experimental import pallas as pl
from jax.experimental.pallas import tpu as pltpu
from jax.experimental.pallas import tpu_sc as plsc

N_NODES = 10000
N_EDGES = 320000
D = 128

NC = 2
NS = 16
W = 80
NB = 3
EDGES_PER_TILE = N_EDGES // (NC * NS)
WINDOWS_PER_TILE = EDGES_PER_TILE // W
NPH = 5
WPP = WINDOWS_PER_TILE // NPH
ROWCHUNK = 80
N_ROW_CHUNKS = N_NODES // ROWCHUNK
CHUNKS_PER_SUBCORE = -(-N_ROW_CHUNKS // NS)


def _sc_aggregate(x, zeros_rows, src2d, dst2d):
    mesh = plsc.VectorSubcoreMesh(core_axis_name="c", subcore_axis_name="s")

    @functools.partial(
        pl.kernel,
        out_type=jax.ShapeDtypeStruct((NC, N_NODES, D), jnp.float32),
        mesh=mesh,
        scratch_types=[
            pltpu.VMEM((WPP, W), jnp.int32),
            pltpu.VMEM((WPP, W), jnp.int32),
            pltpu.VMEM((WPP, W), jnp.int32),
            pltpu.VMEM((WPP, W), jnp.int32),
            pltpu.VMEM((W, D), jnp.float32),
            pltpu.VMEM((W, D), jnp.float32),
            pltpu.VMEM((W, D), jnp.float32),
            pltpu.SemaphoreType.DMA,
            pltpu.SemaphoreType.DMA,
            pltpu.SemaphoreType.DMA,
            pltpu.SemaphoreType.DMA,
            pltpu.SemaphoreType.DMA,
            pltpu.VMEM_SHARED((N_NODES, D), jnp.float32),
        ],
    )
    def k(x_hbm, z_hbm, src_hbm, dst_hbm, out_hbm, src_v0, dst_v0, src_v1,
          dst_v1, rows_0, rows_1, rows_2, gsem_0, gsem_1, gsem_2,
          isem_s, isem_d, acc):
        rows_v = rows_0
        cid = lax.axis_index("c")
        sid = lax.axis_index("s")

        @pl.loop(0, CHUNKS_PER_SUBCORE)
        def _(j):
            k = sid + j * NS

            @pl.when(k < N_ROW_CHUNKS)
            def _():
                pltpu.sync_copy(z_hbm, acc.at[pl.ds(k * ROWCHUNK, ROWCHUNK)])

        wid = cid * NS + sid

        plsc.subcore_barrier()

        def gather_start(src_v, w, buf, gsem):
            pltpu.async_copy(x_hbm.at[src_v.at[w]], buf, gsem)

        def gather_wait(src_v, w, buf, gsem):
            pltpu.make_async_copy(x_hbm.at[src_v.at[w]], buf, gsem).wait()

        def scatter_add(dst_v, w, buf):
            pltpu.sync_copy(buf, acc.at[dst_v.at[w]], add=True)

        bufs = ((rows_0, gsem_0), (rows_1, gsem_1), (rows_2, gsem_2))
        idx = ((src_v0, dst_v0), (src_v1, dst_v1))

        pltpu.sync_copy(src_hbm.at[wid, 0], src_v0)
        pltpu.sync_copy(dst_hbm.at[wid, 0], dst_v0)

        for p in range(NPH):
            src_v, dst_v = idx[p % 2]
            if p + 1 < NPH:
                nsrc_v, ndst_v = idx[(p + 1) % 2]
                pltpu.async_copy(src_hbm.at[wid, p + 1], nsrc_v, isem_s)
                pltpu.async_copy(dst_hbm.at[wid, p + 1], ndst_v, isem_d)

            for i in range(NB):
                gather_start(src_v, i, *bufs[i])

            @pl.loop(0, WPP // NB)
            def _(h, src_v=src_v, dst_v=dst_v):
                w = h * NB
                for i in range(NB):
                    buf, gsem = bufs[i]
                    gather_wait(src_v, w + i, buf, gsem)
                    scatter_add(dst_v, w + i, buf)

                    @pl.when(w + i + NB < WPP)
                    def _():
                        gather_start(src_v, w + i + NB, buf, gsem)

            last = WPP - 1
            buf, gsem = bufs[last % NB]
            gather_wait(src_v, last, buf, gsem)
            scatter_add(dst_v, last, buf)

            if p + 1 < NPH:
                nsrc_v, ndst_v = idx[(p + 1) % 2]
                pltpu.make_async_copy(src_hbm.at[wid, p + 1], nsrc_v, isem_s).wait()
                pltpu.make_async_copy(dst_hbm.at[wid, p + 1], ndst_v, isem_d).wait()

        plsc.subcore_barrier()

        @pl.loop(0, CHUNKS_PER_SUBCORE)
        def _(j):
            k = sid + j * NS

            @pl.when(k < N_ROW_CHUNKS)
            def _():
                base = k * ROWCHUNK
                stage = rows_v.at[pl.ds(0, ROWCHUNK)]
                pltpu.sync_copy(acc.at[pl.ds(base, ROWCHUNK)], stage)
                pltpu.sync_copy(stage, out_hbm.at[cid, pl.ds(base, ROWCHUNK)])

    return k(x, zeros_rows, src2d, dst2d)


def _tc_combine(x, partials, W1T, W2T, b):
    BLK = 1000

    def body(x_ref, p_ref, w1_ref, w2_ref, b_ref, o_ref):
        agg = p_ref[0] + p_ref[1]
        o_ref[...] = (
            jnp.dot(x_ref[...], w1_ref[...], preferred_element_type=jnp.float32)
            + jnp.dot(agg, w2_ref[...], preferred_element_type=jnp.float32)
            + b_ref[...]
        )

    return pl.pallas_call(
        body,
        grid=(N_NODES // BLK,),
        in_specs=[
            pl.BlockSpec((BLK, D), lambda i: (i, 0)),
            pl.BlockSpec((NC, BLK, D), lambda i: (0, i, 0)),
            pl.BlockSpec((D, D), lambda i: (0, 0)),
            pl.BlockSpec((D, D), lambda i: (0, 0)),
            pl.BlockSpec((1, D), lambda i: (0, 0)),
        ],
        out_specs=pl.BlockSpec((BLK, D), lambda i: (i, 0)),
        out_shape=jax.ShapeDtypeStruct((N_NODES, D), jnp.float32),
    )(x, partials, W1T, W2T, b)


def kernel(shape_features, edge_index, W1, b1, W2, b2):
    src2d = edge_index[0].reshape(NC * NS, NPH, WPP, W)
    dst2d = edge_index[1].reshape(NC * NS, NPH, WPP, W)
    zeros_rows = jnp.zeros((ROWCHUNK, D), jnp.float32)
    partials = _sc_aggregate(shape_features, zeros_rows, src2d, dst2d)
    b = (b1 + b2).reshape(1, D)
    return _tc_combine(shape_features, partials, W1.T, W2.T, b)


# speedup vs baseline: 1.2864x; 1.1260x over previous
import functools

import jax
import jax.numpy as jnp
from jax import lax
from jax.experimental import pallas as pl
from jax.experimental.pallas import tpu as pltpu
from jax.experimental.pallas import tpu_sc as plsc

N_NODES = 10000
N_EDGES = 320000
D = 128

NC = 2
NS = 16
W = 80
NB = 3
EDGES_PER_TILE = N_EDGES // (NC * NS)
WINDOWS_PER_TILE = EDGES_PER_TILE // W
NPH = 5
WPP = WINDOWS_PER_TILE // NPH
ROWCHUNK = 80
N_ROW_CHUNKS = N_NODES // ROWCHUNK
CHUNKS_PER_SUBCORE = -(-N_ROW_CHUNKS // NS)


def _sc_aggregate(x, src2d, dst2d):
    mesh = plsc.VectorSubcoreMesh(core_axis_name="c", subcore_axis_name="s")

    @functools.partial(
        pl.kernel,
        out_type=jax.ShapeDtypeStruct((NC, N_NODES, D), jnp.float32),
        mesh=mesh,
        scratch_types=[
            pltpu.VMEM((WPP, W), jnp.int32),
            pltpu.VMEM((WPP, W), jnp.int32),
            pltpu.VMEM((WPP, W), jnp.int32),
            pltpu.VMEM((WPP, W), jnp.int32),
            pltpu.VMEM((W, D), jnp.float32),
            pltpu.VMEM((W, D), jnp.float32),
            pltpu.VMEM((W, D), jnp.float32),
            pltpu.SemaphoreType.DMA,
            pltpu.SemaphoreType.DMA,
            pltpu.SemaphoreType.DMA,
            pltpu.SemaphoreType.DMA,
            pltpu.SemaphoreType.DMA,
            pltpu.VMEM_SHARED((N_NODES, D), jnp.float32),
        ],
    )
    def k(x_hbm, src_hbm, dst_hbm, out_hbm, src_v0, dst_v0, src_v1,
          dst_v1, rows_0, rows_1, rows_2, gsem_0, gsem_1, gsem_2,
          isem_s, isem_d, acc):
        rows_v = rows_0
        cid = lax.axis_index("c")
        sid = lax.axis_index("s")
        wid = cid * NS + sid

        pltpu.async_copy(src_hbm.at[wid, 0], src_v0, isem_s)
        pltpu.async_copy(dst_hbm.at[wid, 0], dst_v0, isem_d)

        zero16 = jnp.zeros((16,), jnp.float32)

        @pl.loop(0, ROWCHUNK)
        def _(r):
            for c in range(0, D, 16):
                rows_v[r, pl.ds(c, 16)] = zero16

        @pl.loop(0, CHUNKS_PER_SUBCORE)
        def _(j):
            k = sid + j * NS

            @pl.when(k < N_ROW_CHUNKS)
            def _():
                pltpu.sync_copy(rows_v, acc.at[pl.ds(k * ROWCHUNK, ROWCHUNK)])

        plsc.subcore_barrier()

        pltpu.make_async_copy(src_hbm.at[wid, 0], src_v0, isem_s).wait()
        pltpu.make_async_copy(dst_hbm.at[wid, 0], dst_v0, isem_d).wait()

        def gather_start(src_v, w, buf, gsem):
            pltpu.async_copy(x_hbm.at[src_v.at[w]], buf, gsem)

        def gather_wait(src_v, w, buf, gsem):
            pltpu.make_async_copy(x_hbm.at[src_v.at[w]], buf, gsem).wait()

        def scatter_add(dst_v, w, buf):
            pltpu.sync_copy(buf, acc.at[dst_v.at[w]], add=True)

        bufs = ((rows_0, gsem_0), (rows_1, gsem_1), (rows_2, gsem_2))
        idx = ((src_v0, dst_v0), (src_v1, dst_v1))

        for p in range(NPH):
            src_v, dst_v = idx[p % 2]
            if p + 1 < NPH:
                nsrc_v, ndst_v = idx[(p + 1) % 2]
                pltpu.async_copy(src_hbm.at[wid, p + 1], nsrc_v, isem_s)
                pltpu.async_copy(dst_hbm.at[wid, p + 1], ndst_v, isem_d)

            for i in range(NB):
                gather_start(src_v, i, *bufs[i])

            @pl.loop(0, WPP // NB)
            def _(h, src_v=src_v, dst_v=dst_v):
                w = h * NB
                for i in range(NB):
                    buf, gsem = bufs[i]
                    gather_wait(src_v, w + i, buf, gsem)
                    scatter_add(dst_v, w + i, buf)

                    @pl.when(w + i + NB < WPP)
                    def _():
                        gather_start(src_v, w + i + NB, buf, gsem)

            last = WPP - 1
            buf, gsem = bufs[last % NB]
            gather_wait(src_v, last, buf, gsem)
            scatter_add(dst_v, last, buf)

            if p + 1 < NPH:
                nsrc_v, ndst_v = idx[(p + 1) % 2]
                pltpu.make_async_copy(src_hbm.at[wid, p + 1], nsrc_v, isem_s).wait()
                pltpu.make_async_copy(dst_hbm.at[wid, p + 1], ndst_v, isem_d).wait()

        plsc.subcore_barrier()

        @pl.loop(0, CHUNKS_PER_SUBCORE)
        def _(j):
            k = sid + j * NS

            @pl.when(k < N_ROW_CHUNKS)
            def _():
                base = k * ROWCHUNK
                stage = rows_v.at[pl.ds(0, ROWCHUNK)]
                pltpu.sync_copy(acc.at[pl.ds(base, ROWCHUNK)], stage)
                pltpu.sync_copy(stage, out_hbm.at[cid, pl.ds(base, ROWCHUNK)])

    return k(x, src2d, dst2d)


def _tc_combine(x, partials, W1T, W2T, b):
    BLK = 1000

    def body(x_ref, p_ref, w1_ref, w2_ref, b_ref, o_ref):
        agg = p_ref[0] + p_ref[1]
        o_ref[...] = (
            jnp.dot(x_ref[...], w1_ref[...], preferred_element_type=jnp.float32)
            + jnp.dot(agg, w2_ref[...], preferred_element_type=jnp.float32)
            + b_ref[...]
        )

    return pl.pallas_call(
        body,
        grid=(N_NODES // BLK,),
        in_specs=[
            pl.BlockSpec((BLK, D), lambda i: (i, 0)),
            pl.BlockSpec((NC, BLK, D), lambda i: (0, i, 0)),
            pl.BlockSpec((D, D), lambda i: (0, 0)),
            pl.BlockSpec((D, D), lambda i: (0, 0)),
            pl.BlockSpec((1, D), lambda i: (0, 0)),
        ],
        out_specs=pl.BlockSpec((BLK, D), lambda i: (i, 0)),
        out_shape=jax.ShapeDtypeStruct((N_NODES, D), jnp.float32),
    )(x, partials, W1T, W2T, b)


def kernel(shape_features, edge_index, W1, b1, W2, b2):
    src2d = edge_index[0].reshape(NC * NS, NPH, WPP, W)
    dst2d = edge_index[1].reshape(NC * NS, NPH, WPP, W)
    partials = _sc_aggregate(shape_features, src2d, dst2d)
    b = (b1 + b2).reshape(1, D)
    return _tc_combine(shape_features, partials, W1.T, W2.T, b)


# speedup vs baseline: 1.3195x; 1.0257x over previous
import functools

import jax
import jax.numpy as jnp
from jax import lax
from jax.experimental import pallas as pl
from jax.experimental.pallas import tpu as pltpu
from jax.experimental.pallas import tpu_sc as plsc

N_NODES = 10000
N_EDGES = 320000
D = 128

NC = 2
NS = 16
W = 80
NB = 3
EDGES_PER_TILE = N_EDGES // (NC * NS)
WINDOWS_PER_TILE = EDGES_PER_TILE // W
NPH = 5
WPP = WINDOWS_PER_TILE // NPH
ROWCHUNK = 80
N_ROW_CHUNKS = N_NODES // ROWCHUNK
CHUNKS_PER_SUBCORE = -(-N_ROW_CHUNKS // NS)


def _sc_aggregate(x, src2d, dst2d):
    mesh = plsc.VectorSubcoreMesh(core_axis_name="c", subcore_axis_name="s")

    @functools.partial(
        pl.kernel,
        out_type=jax.ShapeDtypeStruct((NC, N_NODES, D), jnp.float32),
        mesh=mesh,
        scratch_types=[
            pltpu.VMEM((WPP, W), jnp.int32),
            pltpu.VMEM((WPP, W), jnp.int32),
            pltpu.VMEM((WPP, W), jnp.int32),
            pltpu.VMEM((WPP, W), jnp.int32),
            pltpu.VMEM((W, D), jnp.float32),
            pltpu.VMEM((W, D), jnp.float32),
            pltpu.VMEM((W, D), jnp.float32),
            pltpu.SemaphoreType.DMA,
            pltpu.SemaphoreType.DMA,
            pltpu.SemaphoreType.DMA,
            pltpu.SemaphoreType.DMA,
            pltpu.SemaphoreType.DMA,
            pltpu.VMEM_SHARED((N_NODES, D), jnp.float32),
        ],
    )
    def k(x_hbm, src_hbm, dst_hbm, out_hbm, src_v0, dst_v0, src_v1,
          dst_v1, rows_0, rows_1, rows_2, gsem_0, gsem_1, gsem_2,
          isem_s, isem_d, acc):
        rows_v = rows_0
        cid = lax.axis_index("c")
        sid = lax.axis_index("s")
        wid = cid * NS + sid

        pltpu.async_copy(src_hbm.at[wid, 0], src_v0, isem_s)
        pltpu.async_copy(dst_hbm.at[wid, 0], dst_v0, isem_d)

        zero16 = jnp.zeros((16,), jnp.float32)

        @pl.loop(0, ROWCHUNK)
        def _(r):
            for c in range(0, D, 16):
                rows_v[r, pl.ds(c, 16)] = zero16

        @pl.loop(0, CHUNKS_PER_SUBCORE)
        def _(j):
            k = sid + j * NS

            @pl.when(k < N_ROW_CHUNKS)
            def _():
                pltpu.async_copy(rows_v, acc.at[pl.ds(k * ROWCHUNK, ROWCHUNK)],
                                 gsem_1)

        @pl.loop(0, CHUNKS_PER_SUBCORE)
        def _(j):
            k = sid + j * NS

            @pl.when(k < N_ROW_CHUNKS)
            def _():
                pltpu.make_async_copy(
                    rows_v, acc.at[pl.ds(k * ROWCHUNK, ROWCHUNK)], gsem_1
                ).wait()

        plsc.subcore_barrier()

        pltpu.make_async_copy(src_hbm.at[wid, 0], src_v0, isem_s).wait()
        pltpu.make_async_copy(dst_hbm.at[wid, 0], dst_v0, isem_d).wait()

        def gather_start(src_v, w, buf, gsem):
            pltpu.async_copy(x_hbm.at[src_v.at[w]], buf, gsem)

        def gather_wait(src_v, w, buf, gsem):
            pltpu.make_async_copy(x_hbm.at[src_v.at[w]], buf, gsem).wait()

        def scatter_add(dst_v, w, buf):
            pltpu.sync_copy(buf, acc.at[dst_v.at[w]], add=True)

        bufs = ((rows_0, gsem_0), (rows_1, gsem_1), (rows_2, gsem_2))
        idx = ((src_v0, dst_v0), (src_v1, dst_v1))

        MAIN = WPP - NB - WPP % NB

        for p in range(NPH):
            src_v, dst_v = idx[p % 2]
            if p + 1 < NPH:
                nsrc_v, ndst_v = idx[(p + 1) % 2]
                pltpu.async_copy(src_hbm.at[wid, p + 1], nsrc_v, isem_s)
                pltpu.async_copy(dst_hbm.at[wid, p + 1], ndst_v, isem_d)

            if p == 0:
                for i in range(NB):
                    gather_start(src_v, i, *bufs[i])

            @pl.loop(0, MAIN // NB)
            def _(h, src_v=src_v, dst_v=dst_v):
                w = h * NB
                for i in range(NB):
                    buf, gsem = bufs[i]
                    gather_wait(src_v, w + i, buf, gsem)
                    scatter_add(dst_v, w + i, buf)
                    gather_start(src_v, w + i + NB, buf, gsem)

            if p + 1 < NPH:
                pltpu.make_async_copy(src_hbm.at[wid, p + 1], nsrc_v, isem_s).wait()
                pltpu.make_async_copy(dst_hbm.at[wid, p + 1], ndst_v, isem_d).wait()

            for wt in range(MAIN, WPP):
                buf, gsem = bufs[wt % NB]
                gather_wait(src_v, wt, buf, gsem)
                scatter_add(dst_v, wt, buf)
                if wt + NB < WPP:
                    gather_start(src_v, wt + NB, buf, gsem)
                elif p + 1 < NPH:
                    gather_start(nsrc_v, wt % NB, buf, gsem)

        plsc.subcore_barrier()

        @pl.loop(0, CHUNKS_PER_SUBCORE)
        def _(j):
            k = sid + j * NS

            @pl.when(k < N_ROW_CHUNKS)
            def _():
                base = k * ROWCHUNK
                stage = rows_v.at[pl.ds(0, ROWCHUNK)]
                pltpu.sync_copy(acc.at[pl.ds(base, ROWCHUNK)], stage)
                pltpu.sync_copy(stage, out_hbm.at[cid, pl.ds(base, ROWCHUNK)])

    return k(x, src2d, dst2d)


def _tc_combine(x, partials, W1T, W2T, b):
    BLK = 1000

    def body(x_ref, p_ref, w1_ref, w2_ref, b_ref, o_ref):
        agg = p_ref[0] + p_ref[1]
        o_ref[...] = (
            jnp.dot(x_ref[...], w1_ref[...], preferred_element_type=jnp.float32)
            + jnp.dot(agg, w2_ref[...], preferred_element_type=jnp.float32)
            + b_ref[...]
        )

    return pl.pallas_call(
        body,
        grid=(N_NODES // BLK,),
        in_specs=[
            pl.BlockSpec((BLK, D), lambda i: (i, 0)),
            pl.BlockSpec((NC, BLK, D), lambda i: (0, i, 0)),
            pl.BlockSpec((D, D), lambda i: (0, 0)),
            pl.BlockSpec((D, D), lambda i: (0, 0)),
            pl.BlockSpec((1, D), lambda i: (0, 0)),
        ],
        out_specs=pl.BlockSpec((BLK, D), lambda i: (i, 0)),
        out_shape=jax.ShapeDtypeStruct((N_NODES, D), jnp.float32),
    )(x, partials, W1T, W2T, b)


def kernel(shape_features, edge_index, W1, b1, W2, b2):
    src2d = edge_index[0].reshape(NC * NS, NPH, WPP, W)
    dst2d = edge_index[1].reshape(NC * NS, NPH, WPP, W)
    partials = _sc_aggregate(shape_features, src2d, dst2d)
    b = (b1 + b2).reshape(1, D)
    return _tc_combine(shape_features, partials, W1.T, W2.T, b)
